# baseline (speedup 1.0000x reference)
import functools

import jax
import jax.numpy as jnp
from jax.experimental import pallas as pl
from jax.experimental.pallas import tpu as pltpu


def _matmul_block_kernel(x_ref, adj_ref, o_ref):
    o_ref[...] = jnp.dot(
        adj_ref[...], x_ref[...], preferred_element_type=jnp.float32
    )


@functools.partial(jax.jit, static_argnames=("block_m",))
def _sum_aggregate(x, adj, block_m=400):
    m, k = adj.shape
    _, n = x.shape
    grid = (m // block_m,)
    return pl.pallas_call(
        _matmul_block_kernel,
        grid=grid,
        in_specs=[
            pl.BlockSpec((k, n), lambda i: (0, 0)),
            pl.BlockSpec((block_m, k), lambda i: (i, 0)),
        ],
        out_specs=pl.BlockSpec((block_m, n), lambda i: (i, 0)),
        out_shape=jax.ShapeDtypeStruct((m, n), jnp.float32),
        compiler_params=pltpu.CompilerParams(
            dimension_semantics=("arbitrary",),
            vmem_limit_bytes=110 * 1024 * 1024,
        ),
    )(x, adj)


def kernel(x, adj):
    return _sum_aggregate(x, adj)


# speedup vs baseline: 1.0050x; 1.0050x over previous
import functools

import jax
import jax.numpy as jnp
from jax.experimental import pallas as pl
from jax.experimental.pallas import tpu as pltpu


def _matmul_block_kernel(x_ref, adj_ref, o_ref):
    o_ref[...] = jnp.dot(
        adj_ref[...], x_ref[...], preferred_element_type=jnp.float32
    )


@functools.partial(jax.jit, static_argnames=("block_m",))
def _sum_aggregate(x, adj, block_m=200):
    m, k = adj.shape
    _, n = x.shape
    grid = (m // block_m,)
    return pl.pallas_call(
        _matmul_block_kernel,
        grid=grid,
        in_specs=[
            pl.BlockSpec((k, n), lambda i: (0, 0)),
            pl.BlockSpec((block_m, k), lambda i: (i, 0)),
        ],
        out_specs=pl.BlockSpec((block_m, n), lambda i: (i, 0)),
        out_shape=jax.ShapeDtypeStruct((m, n), jnp.float32),
        compiler_params=pltpu.CompilerParams(
            dimension_semantics=("arbitrary",),
            vmem_limit_bytes=110 * 1024 * 1024,
        ),
    )(x, adj)


def kernel(x, adj):
    return _sum_aggregate(x, adj)
